# baseline (speedup 1.0000x reference)
import functools

import jax
import jax.numpy as jnp
from jax import lax
from jax.experimental import pallas as pl
from jax.experimental.pallas import tpu as pltpu
from jax.experimental.pallas import tpu_sc as plsc

NC, NS, L = 2, 16, 16
NW = NC * NS
R, C = 128, 32768
K = 16
RPW = R // NW
HC = C // 2
BLK = 32
NBLK = C // (BLK * L)

_MESH = plsc.VectorSubcoreMesh(
    core_axis_name="c", subcore_axis_name="s", num_cores=NC, num_subcores=NS
)


def _lane_iota():
    return lax.iota(jnp.int32, L)


def _lane_sum(vec, scratch):
    lane = _lane_iota()
    acc = vec
    for shift in (8, 4, 2, 1):
        scratch[...] = acc
        idx = lax.bitwise_xor(lane, jnp.full((L,), shift, jnp.int32))
        acc = acc + plsc.load_gather(scratch, [idx])
    return acc


def _score_from_var(v):
    v = jnp.maximum(v, jnp.full((L,), 1e-30, jnp.float32))
    bits = lax.bitcast_convert_type(v, jnp.int32)
    seed = jnp.full((L,), 0x5F3759DF, jnp.int32) - lax.shift_right_arithmetic(
        bits, jnp.full((L,), 1, jnp.int32)
    )
    y = lax.bitcast_convert_type(seed, jnp.float32)
    for _ in range(3):
        y = y * (1.5 - 0.5 * v * y * y)
    std = v * y
    std = 0.5 * (std + v / std)
    realism = jnp.where(
        std < 0.01,
        std * 10.0,
        jnp.where(std > 0.5, 0.5 / std, 1.0 - jnp.abs(std - 0.1)),
    )
    return realism + 0.15


@functools.partial(
    pl.kernel,
    out_type=jax.ShapeDtypeStruct((NW, L), jnp.float32),
    mesh=_MESH,
    scratch_types=[
        pltpu.VMEM((C,), jnp.float32),
        pltpu.VMEM((C,), jnp.float32),
        pltpu.VMEM((L,), jnp.float32),
        pltpu.VMEM((L,), jnp.float32),
        pltpu.SemaphoreType.DMA,
        pltpu.SemaphoreType.DMA,
    ],
    compiler_params=pltpu.CompilerParams(needs_layout_passes=False),
)
def _score_stage(chunks_hbm, out_hbm, buf0, buf1, red, svmem, sem0, sem1):
    wid = lax.axis_index("s") * NC + lax.axis_index("c")
    row0 = wid * RPW
    bufs = (buf0, buf1)
    sems = (sem0, sem1)
    zeros = jnp.zeros((L,), jnp.float32)
    lane = _lane_iota()

    copies = [None] * RPW
    copies[0] = pltpu.async_copy(chunks_hbm.at[row0], buf0, sem0)

    score_vec = zeros
    for r in range(RPW):
        buf = bufs[r % 2]
        copies[r].wait()
        if r + 1 < RPW:
            copies[r + 1] = pltpu.async_copy(
                chunks_hbm.at[row0 + r + 1], bufs[(r + 1) % 2], sems[(r + 1) % 2]
            )

        def blk(b, carry, buf=buf):
            s, q = carry
            off = b * (BLK * L)
            x = buf[pl.ds(off, L)]
            ls = x
            lq = x * x
            for i in range(1, BLK):
                x = buf[pl.ds(off + i * L, L)]
                ls = ls + x
                lq = lq + x * x
            return (s + ls, q + lq)

        s, q = lax.fori_loop(0, NBLK, blk, (zeros, zeros))
        ssum = _lane_sum(s, red)
        qsum = _lane_sum(q, red)
        var = (qsum - ssum * ssum * (1.0 / C)) * (1.0 / (C - 1))
        score_r = _score_from_var(var)
        score_vec = jnp.where(lane == r, score_r, score_vec)

    svmem[...] = score_vec
    pltpu.sync_copy(svmem, out_hbm.at[wid])


@functools.partial(
    pl.kernel,
    out_type=(
        jax.ShapeDtypeStruct((NW, HC), jnp.float32),
        jax.ShapeDtypeStruct((K,), jnp.float32),
    ),
    mesh=_MESH,
    scratch_types=[
        pltpu.VMEM((NW, L), jnp.float32),
        pltpu.VMEM((K,), jnp.int32),
        pltpu.VMEM((K,), jnp.float32),
        pltpu.VMEM((1,), jnp.int32),
        pltpu.VMEM((1, HC), jnp.float32),
        pltpu.SemaphoreType.DMA,
    ],
    compiler_params=pltpu.CompilerParams(needs_layout_passes=False),
)
def _topk_gather_stage(chunks2_hbm, scores_hbm, out_hbm, oscores_hbm,
                       sraw, tidx, tsc, idxv, halfbuf, sem):
    wid = lax.axis_index("s") * NC + lax.axis_index("c")
    lane = _lane_iota()

    pltpu.sync_copy(scores_hbm, sraw)

    svecs = []
    for v in range(8):
        jv = lane + (16 * v)
        svecs.append(
            plsc.load_gather(
                sraw,
                [
                    lax.shift_right_arithmetic(jv, jnp.full((L,), 2, jnp.int32)),
                    lax.bitwise_and(jv, jnp.full((L,), RPW - 1, jnp.int32)),
                ],
            )
        )

    one = jnp.full((L,), 1, jnp.int32)
    zero = jnp.full((L,), 0, jnp.int32)

    def body(j, cnts):
        sj = plsc.load_gather(
            sraw, [jnp.full((L,), j // RPW, jnp.int32),
                   jnp.full((L,), j % RPW, jnp.int32)]
        )
        jv = jnp.full((L,), j, jnp.int32)
        out = []
        for v in range(8):
            gidx = lane + (16 * v)
            gt = sj > svecs[v]
            eqlt = jnp.logical_and(sj == svecs[v], jv < gidx)
            out.append(cnts[v] + jnp.where(gt, one, zero) + jnp.where(eqlt, one, zero))
        return tuple(out)

    cnts = lax.fori_loop(0, R, body, tuple([zero] * 8))

    kmax = jnp.full((L,), K - 1, jnp.int32)
    for v in range(8):
        gidx = lane + (16 * v)
        sel = cnts[v] < K
        ridx = jnp.minimum(cnts[v], kmax)
        plsc.store_scatter(tidx, [ridx], gidx, mask=sel)
        plsc.store_scatter(tsc, [ridx], svecs[v], mask=sel)

    @pl.when(wid == 0)
    def _():
        pltpu.sync_copy(tsc, oscores_hbm)

    r = wid // 2
    h = wid % 2
    rowvec = plsc.load_gather(tidx, [jnp.full((L,), r, jnp.int32)])
    subrow = rowvec + rowvec + h
    plsc.store_scatter(idxv, [zero], subrow, mask=lane == 0)
    pltpu.async_copy(chunks2_hbm.at[idxv], halfbuf, sem).wait()
    pltpu.sync_copy(halfbuf, out_hbm.at[pl.ds(wid, 1)])


def kernel(chunks, regime_probs, keep_top_k):
    del regime_probs
    scores32 = _score_stage(chunks)
    chunks2 = chunks.reshape(2 * R, HC)
    pruned32, top_scores = _topk_gather_stage(chunks2, scores32)
    pruned = pruned32.reshape(K, C)
    top_scores = top_scores + 0.0 * jnp.asarray(keep_top_k, top_scores.dtype)
    return (pruned, top_scores)


# speedup vs baseline: 1.2813x; 1.2813x over previous
import functools

import jax
import jax.numpy as jnp
from jax import lax
from jax.experimental import pallas as pl
from jax.experimental.pallas import tpu as pltpu
from jax.experimental.pallas import tpu_sc as plsc

NC, NS, L = 2, 16, 16
NW = NC * NS
R, C = 128, 32768
K = 16
RPW = R // NW
HC = C // 2
BLK = 32
NBLK = C // (BLK * L)

_MESH = plsc.VectorSubcoreMesh(
    core_axis_name="c", subcore_axis_name="s", num_cores=NC, num_subcores=NS
)


def _lane_iota():
    return lax.iota(jnp.int32, L)


def _lane_sum(vec, scratch):
    lane = _lane_iota()
    acc = vec
    for shift in (8, 4, 2, 1):
        scratch[...] = acc
        idx = lax.bitwise_xor(lane, jnp.full((L,), shift, jnp.int32))
        acc = acc + plsc.load_gather(scratch, [idx])
    return acc


def _score_from_var(v):
    v = jnp.maximum(v, jnp.full((L,), 1e-30, jnp.float32))
    bits = lax.bitcast_convert_type(v, jnp.int32)
    seed = jnp.full((L,), 0x5F3759DF, jnp.int32) - lax.shift_right_arithmetic(
        bits, jnp.full((L,), 1, jnp.int32)
    )
    y = lax.bitcast_convert_type(seed, jnp.float32)
    for _ in range(3):
        y = y * (1.5 - 0.5 * v * y * y)
    std = v * y
    std = 0.5 * (std + v / std)
    realism = jnp.where(
        std < 0.01,
        std * 10.0,
        jnp.where(std > 0.5, 0.5 / std, 1.0 - jnp.abs(std - 0.1)),
    )
    return realism + 0.15


@functools.partial(
    pl.kernel,
    out_type=jax.ShapeDtypeStruct((NW, L), jnp.float32),
    mesh=_MESH,
    scratch_types=[
        pltpu.VMEM((C,), jnp.float32),
        pltpu.VMEM((C,), jnp.float32),
        pltpu.VMEM((L,), jnp.float32),
        pltpu.VMEM((L,), jnp.float32),
        pltpu.SemaphoreType.DMA,
        pltpu.SemaphoreType.DMA,
    ],
    compiler_params=pltpu.CompilerParams(needs_layout_passes=False),
)
def _score_stage(chunks_hbm, out_hbm, buf0, buf1, red, svmem, sem0, sem1):
    wid = lax.axis_index("s") * NC + lax.axis_index("c")
    row0 = wid * RPW
    bufs = (buf0, buf1)
    sems = (sem0, sem1)
    zeros = jnp.zeros((L,), jnp.float32)
    lane = _lane_iota()

    copies = [None] * RPW
    copies[0] = pltpu.async_copy(chunks_hbm.at[row0], buf0, sem0)

    score_vec = zeros
    for r in range(RPW):
        buf = bufs[r % 2]
        copies[r].wait()
        if r + 1 < RPW:
            copies[r + 1] = pltpu.async_copy(
                chunks_hbm.at[row0 + r + 1], bufs[(r + 1) % 2], sems[(r + 1) % 2]
            )

        def blk(b, carry, buf=buf):
            s, q = carry
            off = b * (BLK * L)
            x = buf[pl.ds(off, L)]
            ls = x
            lq = x * x
            for i in range(1, BLK):
                x = buf[pl.ds(off + i * L, L)]
                ls = ls + x
                lq = lq + x * x
            return (s + ls, q + lq)

        s, q = lax.fori_loop(0, NBLK, blk, (zeros, zeros))
        ssum = _lane_sum(s, red)
        qsum = _lane_sum(q, red)
        var = (qsum - ssum * ssum * (1.0 / C)) * (1.0 / (C - 1))
        score_r = _score_from_var(var)
        score_vec = jnp.where(lane == r, score_r, score_vec)

    svmem[...] = score_vec
    pltpu.sync_copy(svmem, out_hbm.at[wid])


_KEY_BASE = 0x3E19999A


@functools.partial(
    pl.kernel,
    out_type=(
        jax.ShapeDtypeStruct((K, C), jnp.float32),
        jax.ShapeDtypeStruct((K,), jnp.float32),
    ),
    mesh=_MESH,
    scratch_types=[
        pltpu.VMEM((NW, L), jnp.float32),
        pltpu.VMEM((K,), jnp.int32),
        pltpu.VMEM((K,), jnp.float32),
        pltpu.VMEM((1,), jnp.int32),
        pltpu.VMEM((1, HC), jnp.float32),
        pltpu.SemaphoreType.DMA,
    ],
    compiler_params=pltpu.CompilerParams(needs_layout_passes=False),
)
def _topk_gather_stage(chunks_hbm, scores_hbm, out_hbm, oscores_hbm,
                       sraw, tidx, tsc, idxv, halfbuf, sem):
    wid = lax.axis_index("s") * NC + lax.axis_index("c")
    lane = _lane_iota()
    zero = jnp.full((L,), 0, jnp.int32)

    pltpu.sync_copy(scores_hbm, sraw)

    pairs = []
    for v in range(8):
        jv = lane + (16 * v)
        sv = plsc.load_gather(
            sraw,
            [
                lax.shift_right_arithmetic(jv, jnp.full((L,), 2, jnp.int32)),
                lax.bitwise_and(jv, jnp.full((L,), RPW - 1, jnp.int32)),
            ],
        )
        bits = lax.bitcast_convert_type(sv, jnp.uint32)
        diff = bits - jnp.full((L,), _KEY_BASE, jnp.uint32)
        key = lax.bitwise_or(
            lax.shift_left(diff, jnp.full((L,), 7, jnp.uint32)),
            lax.bitcast_convert_type(jnp.full((L,), 127, jnp.int32) - jv,
                                     jnp.uint32),
        )
        k_s, v_s = plsc.sort_key_val(key, jv, descending=True)
        pairs.append((k_s, v_s))

    def merge(a, b):
        ka, va = a
        kb, vb = b
        kr = lax.rev(kb, (0,))
        vr = lax.rev(vb, (0,))
        m = ka >= kr
        kk = jnp.where(m, ka, kr)
        vv = jnp.where(m, va, vr)
        return plsc.sort_key_val(kk, vv, descending=True)

    while len(pairs) > 1:
        pairs = [merge(pairs[i], pairs[i + 1]) for i in range(0, len(pairs), 2)]
    _, top_rows = pairs[0]

    tidx[...] = top_rows
    tsc[...] = plsc.load_gather(
        sraw,
        [
            lax.shift_right_arithmetic(top_rows, jnp.full((L,), 2, jnp.int32)),
            lax.bitwise_and(top_rows, jnp.full((L,), RPW - 1, jnp.int32)),
        ],
    )

    @pl.when(wid == 0)
    def _():
        pltpu.sync_copy(tsc, oscores_hbm)

    r = wid // 2
    h = wid % 2
    rowvec = plsc.load_gather(tidx, [jnp.full((L,), r, jnp.int32)])
    plsc.store_scatter(idxv, [zero], rowvec, mask=lane == 0)
    colbase = h * HC
    pltpu.async_copy(chunks_hbm.at[idxv, pl.ds(colbase, HC)], halfbuf, sem).wait()
    pltpu.sync_copy(halfbuf, out_hbm.at[pl.ds(r, 1), pl.ds(colbase, HC)])


def kernel(chunks, regime_probs, keep_top_k):
    del regime_probs, keep_top_k
    scores32 = _score_stage(chunks)
    return _topk_gather_stage(chunks, scores32)


# speedup vs baseline: 1.6514x; 1.2889x over previous
import functools

import jax
import jax.numpy as jnp
from jax import lax
from jax.experimental import pallas as pl
from jax.experimental.pallas import tpu as pltpu
from jax.experimental.pallas import tpu_sc as plsc

NC, NS, L = 2, 16, 16
NW = NC * NS
R, C = 128, 32768
K = 16
HC = C // 2
BR = 16

_MESH = plsc.VectorSubcoreMesh(
    core_axis_name="c", subcore_axis_name="s", num_cores=NC, num_subcores=NS
)

_KEY_BASE = 0x3E19999A


def _tc_score_body(x_ref, out_ref):
    x = x_ref[...]
    s = jnp.sum(x, axis=1)
    q = jnp.sum(x * x, axis=1)
    var = (q - s * s * (1.0 / C)) * (1.0 / (C - 1))
    std = jnp.sqrt(jnp.maximum(var, 0.0))
    realism = jnp.where(
        std < 0.01,
        std * 10.0,
        jnp.where(std > 0.5, 0.5 / std, 1.0 - jnp.abs(std - 0.1)),
    )
    out_ref[...] = (realism + 0.15).reshape(1, 1, BR)


_score_tc = pl.pallas_call(
    _tc_score_body,
    grid=(R // BR,),
    in_specs=[pl.BlockSpec((BR, C), lambda i: (i, 0))],
    out_specs=pl.BlockSpec((1, 1, BR), lambda i: (i, 0, 0)),
    out_shape=jax.ShapeDtypeStruct((R // BR, 1, BR), jnp.float32),
    compiler_params=pltpu.CompilerParams(dimension_semantics=("arbitrary",)),
)


def _lane_iota():
    return lax.iota(jnp.int32, L)


def _gather_scores(sraw, rows):
    return plsc.load_gather(
        sraw,
        [
            lax.shift_right_arithmetic(rows, jnp.full((L,), 4, jnp.int32)),
            jnp.full((L,), 0, jnp.int32),
            lax.bitwise_and(rows, jnp.full((L,), BR - 1, jnp.int32)),
        ],
    )


@functools.partial(
    pl.kernel,
    out_type=(
        jax.ShapeDtypeStruct((K, C), jnp.float32),
        jax.ShapeDtypeStruct((K,), jnp.float32),
    ),
    mesh=_MESH,
    scratch_types=[
        pltpu.VMEM((R // BR, 1, BR), jnp.float32),
        pltpu.VMEM((K,), jnp.int32),
        pltpu.VMEM((K,), jnp.float32),
        pltpu.VMEM((1,), jnp.int32),
        pltpu.VMEM((1, HC), jnp.float32),
        pltpu.SemaphoreType.DMA,
    ],
    compiler_params=pltpu.CompilerParams(needs_layout_passes=False),
)
def _topk_gather_stage(chunks_hbm, scores_hbm, out_hbm, oscores_hbm,
                       sraw, tidx, tsc, idxv, halfbuf, sem):
    wid = lax.axis_index("s") * NC + lax.axis_index("c")
    lane = _lane_iota()
    zero = jnp.full((L,), 0, jnp.int32)

    pltpu.sync_copy(scores_hbm, sraw)

    pairs = []
    for v in range(8):
        jv = lane + (16 * v)
        sv = _gather_scores(sraw, jv)
        bits = lax.bitcast_convert_type(sv, jnp.uint32)
        diff = bits - jnp.full((L,), _KEY_BASE, jnp.uint32)
        key = lax.bitwise_or(
            lax.shift_left(diff, jnp.full((L,), 7, jnp.uint32)),
            lax.bitcast_convert_type(jnp.full((L,), 127, jnp.int32) - jv,
                                     jnp.uint32),
        )
        pairs.append(plsc.sort_key_val(key, jv, descending=True))

    def merge(a, b):
        ka, va = a
        kb, vb = b
        kr = lax.rev(kb, (0,))
        vr = lax.rev(vb, (0,))
        m = ka >= kr
        kk = jnp.where(m, ka, kr)
        vv = jnp.where(m, va, vr)
        return plsc.sort_key_val(kk, vv, descending=True)

    while len(pairs) > 1:
        pairs = [merge(pairs[i], pairs[i + 1]) for i in range(0, len(pairs), 2)]
    _, top_rows = pairs[0]

    tidx[...] = top_rows
    tsc[...] = _gather_scores(sraw, top_rows)

    @pl.when(wid == 0)
    def _():
        pltpu.sync_copy(tsc, oscores_hbm)

    r = wid // 2
    h = wid % 2
    rowvec = plsc.load_gather(tidx, [jnp.full((L,), r, jnp.int32)])
    plsc.store_scatter(idxv, [zero], rowvec, mask=lane == 0)
    colbase = h * HC
    pltpu.async_copy(chunks_hbm.at[idxv, pl.ds(colbase, HC)], halfbuf, sem).wait()
    pltpu.sync_copy(halfbuf, out_hbm.at[pl.ds(r, 1), pl.ds(colbase, HC)])


def kernel(chunks, regime_probs, keep_top_k):
    del regime_probs, keep_top_k
    scores = _score_tc(chunks)
    return _topk_gather_stage(chunks, scores)
